# baseline (speedup 1.0000x reference)
import functools

import jax
import jax.numpy as jnp
from jax import lax
from jax.experimental import pallas as pl
from jax.experimental.pallas import tpu as pltpu
from jax.experimental.pallas import tpu_sc as plsc

_CHUNK = 128
_NBUF = 3
_L = 16


@functools.cache
def _make_lookup(K, D, B, T, NC, NS):
    N = B * T
    NW = NC * NS
    per_w = N // NW
    nch = per_w // _CHUNK
    w_per_row = T // per_w
    mesh = plsc.VectorSubcoreMesh(core_axis_name="c", subcore_axis_name="s")

    @functools.partial(
        pl.kernel,
        mesh=mesh,
        out_type=jax.ShapeDtypeStruct((B, T, D), jnp.float32),
        scratch_types=[
            pltpu.VMEM((per_w,), jnp.int32),
            pltpu.VMEM((_NBUF, _CHUNK), jnp.float32),
            pltpu.VMEM((_NBUF, _CHUNK, D), jnp.float32),
        ] + [pltpu.SemaphoreType.DMA] * (2 * _NBUF),
    )
    def lookup(sum_hbm, usage_hbm, idx_hbm, out_hbm,
               idx_v, usage_v, rows_v, g0, g1, g2, s0, s1, s2):
        wid = lax.axis_index("s") * NC + lax.axis_index("c")
        brow = wid // w_per_row
        tcol = (wid % w_per_row) * per_w
        gsem = (g0, g1, g2)
        ssem = (s0, s1, s2)

        def gather_rows(c, b):
            return pltpu.make_async_copy(
                sum_hbm.at[idx_v.at[pl.ds(c * _CHUNK, _CHUNK)]],
                rows_v.at[b], gsem[b])

        def gather_usage(c, b):
            return pltpu.make_async_copy(
                usage_hbm.at[idx_v.at[pl.ds(c * _CHUNK, _CHUNK)]],
                usage_v.at[b], gsem[b])

        def start_gathers(c, b):
            gather_rows(c, b).start()
            gather_usage(c, b).start()

        def wait_gathers(b):
            gather_rows(0, b).wait()
            gather_usage(0, b).wait()

        def scatter_copy(c, b):
            return pltpu.make_async_copy(
                rows_v.at[b],
                out_hbm.at[brow, pl.ds(tcol + c * _CHUNK, _CHUNK)], ssem[b])

        def scale_chunk(b):
            def scale_body(g, carry2):
                r0 = g * _L
                scales = 1.0 / usage_v[b, pl.ds(r0, _L)]
                for j in range(_L):
                    for k in range(D // _L):
                        sl = pl.ds(k * _L, _L)
                        rows_v[b, r0 + j, sl] = (
                            rows_v[b, r0 + j, sl] * scales[j])
                return carry2
            lax.fori_loop(0, _CHUNK // _L, scale_body, None)

        pltpu.sync_copy(idx_hbm.at[brow, pl.ds(tcol, per_w)], idx_v)
        start_gathers(0, 0)
        start_gathers(1, 1)

        def ring_body(i, carry):
            for b in range(_NBUF):
                c = i * _NBUF + b
                wait_gathers(b)
                scale_chunk(b)
                scatter_copy(c, b).start()

                bn = (b + 2) % _NBUF

                @pl.when(jnp.logical_and(c + 2 < nch, c >= 1))
                def _():
                    scatter_copy(0, bn).wait()
                    start_gathers(c + 2, bn)

                @pl.when(jnp.logical_and(c + 2 < nch, c < 1))
                def _():
                    start_gathers(c + 2, bn)
            return carry
        lax.fori_loop(0, (nch - 1) // _NBUF, ring_body, None)

        bl = (nch - 1) % _NBUF
        wait_gathers(bl)
        scale_chunk(bl)
        scatter_copy(nch - 1, bl).start()

        for b in range(_NBUF):
            scatter_copy(0, b).wait()

    return lookup


def kernel(indices, cluster_usage, embedding_sum):
    K, D = embedding_sum.shape
    B, T = indices.shape
    N = B * T

    info = plsc.get_sparse_core_info()
    NC, NS = info.num_cores, info.num_subcores
    NW = NC * NS
    per_w = N // NW
    assert N % (NW * _CHUNK) == 0 and D % _L == 0
    assert (per_w // _CHUNK - 1) % _NBUF == 0 and T % per_w == 0

    return _make_lookup(K, D, B, T, NC, NS)(
        embedding_sum, cluster_usage, indices.astype(jnp.int32))


# speedup vs baseline: 1.0378x; 1.0378x over previous
import functools

import jax
import jax.numpy as jnp
from jax import lax
from jax.experimental import pallas as pl
from jax.experimental.pallas import tpu as pltpu
from jax.experimental.pallas import tpu_sc as plsc

_CHUNK = 128
_NBUF = 2
_L = 16


@functools.cache
def _make_lookup(K, D, B, T, NC, NS):
    N = B * T
    NW = NC * NS
    per_w = N // NW
    nch = per_w // _CHUNK
    w_per_row = T // per_w
    mesh = plsc.VectorSubcoreMesh(core_axis_name="c", subcore_axis_name="s")

    @functools.partial(
        pl.kernel,
        mesh=mesh,
        out_type=jax.ShapeDtypeStruct((B, T, D), jnp.float32),
        scratch_types=[
            pltpu.VMEM((per_w,), jnp.int32),
            pltpu.VMEM((_NBUF, _CHUNK), jnp.float32),
            pltpu.VMEM((_NBUF, _CHUNK, D), jnp.float32),
        ] + [pltpu.SemaphoreType.DMA] * (2 * _NBUF),
    )
    def lookup(sum_hbm, usage_hbm, idx_hbm, out_hbm,
               idx_v, usage_v, rows_v, g0, g1, s0, s1):
        wid = lax.axis_index("s") * NC + lax.axis_index("c")
        brow = wid // w_per_row
        tcol = (wid % w_per_row) * per_w
        gsem = (g0, g1)
        ssem = (s0, s1)

        def gather_rows(c, b):
            return pltpu.make_async_copy(
                sum_hbm.at[idx_v.at[pl.ds(c * _CHUNK, _CHUNK)]],
                rows_v.at[b], gsem[b])

        def gather_usage(c, b):
            return pltpu.make_async_copy(
                usage_hbm.at[idx_v.at[pl.ds(c * _CHUNK, _CHUNK)]],
                usage_v.at[b], gsem[b])

        def start_gathers(c, b):
            gather_rows(c, b).start()
            gather_usage(c, b).start()

        def wait_gathers(b):
            gather_rows(0, b).wait()
            gather_usage(0, b).wait()

        def scatter_copy(c, b):
            return pltpu.make_async_copy(
                rows_v.at[b],
                out_hbm.at[brow, pl.ds(tcol + c * _CHUNK, _CHUNK)], ssem[b])

        def scale_chunk(b):
            def scale_body(g, carry2):
                r0 = g * _L
                scales = 1.0 / usage_v[b, pl.ds(r0, _L)]
                for j in range(_L):
                    for k in range(D // _L):
                        sl = pl.ds(k * _L, _L)
                        rows_v[b, r0 + j, sl] = (
                            rows_v[b, r0 + j, sl] * scales[j])
                return carry2
            lax.fori_loop(0, _CHUNK // _L, scale_body, None)

        pltpu.sync_copy(idx_hbm.at[brow, pl.ds(tcol, per_w)], idx_v)
        for b in range(_NBUF):
            start_gathers(b, b)

        def ring_body(i, carry):
            for b in range(_NBUF):
                c = i * _NBUF + b
                wait_gathers(b)
                scale_chunk(b)
                scatter_copy(c, b).start()

                @pl.when(c + _NBUF < nch)
                def _():
                    scatter_copy(0, b).wait()
                    start_gathers(c + _NBUF, b)
            return carry
        lax.fori_loop(0, nch // _NBUF, ring_body, None)

        for b in range(_NBUF):
            scatter_copy(0, b).wait()

    return lookup


def kernel(indices, cluster_usage, embedding_sum):
    K, D = embedding_sum.shape
    B, T = indices.shape
    N = B * T

    info = plsc.get_sparse_core_info()
    NC, NS = info.num_cores, info.num_subcores
    NW = NC * NS
    per_w = N // NW
    assert N % (NW * _CHUNK) == 0 and D % _L == 0
    assert (per_w // _CHUNK) % _NBUF == 0 and T % per_w == 0

    return _make_lookup(K, D, B, T, NC, NS)(
        embedding_sum, cluster_usage, indices.astype(jnp.int32))
